# baseline (speedup 1.0000x reference)
import functools

import jax
import jax.numpy as jnp
from jax import lax
from jax.experimental import pallas as pl
from jax.experimental.pallas import tpu as pltpu
from jax.experimental.pallas import tpu_sc as plsc

_HI = jax.lax.Precision.HIGHEST
_SEG = 224


def _topk_kernel(x_ref, e_ref, i1_ref, i2_ref):
    e = e_ref[...]
    en2 = jnp.sum(e * e, axis=0, keepdims=True)
    B = x_ref.shape[0]
    N = x_ref.shape[2]
    K = e.shape[1]
    for b in range(B):
        xb = x_ref[b]
        prod = lax.dot_general(
            xb, e, (((0,), (0,)), ((), ())),
            preferred_element_type=jnp.float32, precision=_HI)
        scores = en2 - 2.0 * prod
        kiota = lax.broadcasted_iota(jnp.int32, (N, K), 1)
        i1 = jnp.argmin(scores, axis=-1).astype(jnp.int32)
        masked = jnp.where(kiota == i1[:, None], jnp.inf, scores)
        i2 = jnp.argmin(masked, axis=-1).astype(jnp.int32)
        i1_ref[b] = i1
        i2_ref[b] = i2


def _refine_kernel(x_ref, g_ref, i1_ref, i2_ref, out_ref, idx_ref):
    B = x_ref.shape[0]
    N = x_ref.shape[2]
    for b in range(B):
        xb = x_ref[b]
        D = xb.shape[0]
        g1 = g_ref[b].T[:D, :N]
        g2 = g_ref[B + b].T[:D, :N]
        d1 = jnp.sqrt(_exact_d2(xb, g1))
        d2 = jnp.sqrt(_exact_d2(xb, g2))
        i1r = i1_ref[b][None, :]
        i2r = i2_ref[b][None, :]
        swap = (d2 < d1) | ((d2 == d1) & (i2r < i1r))
        idx_ref[b] = jnp.where(swap, i2r, i1r)[0]
        out_ref[b] = jnp.where(swap, g2, g1)


def _exact_d2(xb, g):
    diff = xb - g
    t = diff * diff
    p = t[0:8, :]
    for gi in range(1, 8):
        p = p + t[8 * gi:8 * gi + 8, :]
    a0 = p[0:1, :] + p[4:5, :]
    a2 = p[2:3, :] + p[6:7, :]
    a1 = p[1:2, :] + p[5:6, :]
    a3 = p[3:4, :] + p[7:8, :]
    return (a0 + a2) + (a1 + a3)


def _make_sc_gather(n_rows, d):
    info = plsc.get_sparse_core_info()
    nw = info.num_cores * info.num_subcores
    rows_per_w = n_rows // nw
    mesh = plsc.VectorSubcoreMesh(core_axis_name="c", subcore_axis_name="s")

    @functools.partial(
        pl.kernel, mesh=mesh,
        out_type=jax.ShapeDtypeStruct((n_rows, d), jnp.float32),
        scratch_types=[
            pltpu.VMEM((rows_per_w,), jnp.int32),
            pltpu.VMEM((rows_per_w, d), jnp.float32),
            pltpu.SemaphoreType.DMA,
        ],
    )
    def sc_gather(table_hbm, idx_hbm, out_hbm, idx_v, rows_v, sem):
        wid = lax.axis_index("s") * info.num_cores + lax.axis_index("c")
        base = wid * rows_per_w
        pltpu.sync_copy(idx_hbm.at[pl.ds(base, rows_per_w)], idx_v)
        pltpu.async_copy(table_hbm.at[idx_v], rows_v, sem).wait()
        pltpu.sync_copy(rows_v, out_hbm.at[pl.ds(base, rows_per_w)])

    return sc_gather


def kernel(feature_vectors, embeddings):
    B, D, N = feature_vectors.shape
    K = embeddings.shape[1]

    i1, i2 = pl.pallas_call(
        _topk_kernel,
        out_shape=(
            jax.ShapeDtypeStruct((B, N), jnp.int32),
            jax.ShapeDtypeStruct((B, N), jnp.int32),
        ),
    )(feature_vectors, embeddings)

    idx_pad = jnp.zeros((2 * B, _SEG), jnp.int32)
    idx_pad = idx_pad.at[:B, :N].set(i1).at[B:, :N].set(i2)
    n_rows = 2 * B * _SEG

    et = jnp.zeros((K, 128), jnp.float32).at[:, :D].set(embeddings.T)
    gathered = _make_sc_gather(n_rows, 128)(et, idx_pad.reshape(n_rows))
    gathered = gathered.reshape(2 * B, _SEG, 128)

    out, idx = pl.pallas_call(
        _refine_kernel,
        out_shape=(
            jax.ShapeDtypeStruct((B, D, N), jnp.float32),
            jax.ShapeDtypeStruct((B, N), jnp.int32),
        ),
    )(feature_vectors, gathered, i1, i2)
    return out, idx


# speedup vs baseline: 4.9001x; 4.9001x over previous
import jax
import jax.numpy as jnp
from jax import lax
from jax.experimental import pallas as pl


def _split3(a):
    hi = a.astype(jnp.bfloat16)
    r = a - hi.astype(jnp.float32)
    mid = r.astype(jnp.bfloat16)
    lo = (r - mid.astype(jnp.float32)).astype(jnp.bfloat16)
    return hi, mid, lo


def _exact_d2(xb, g):
    diff = xb - g
    t = diff * diff
    p = t[0:8, :]
    for gi in range(1, 8):
        p = p + t[8 * gi:8 * gi + 8, :]
    a0 = p[0:1, :] + p[4:5, :]
    a2 = p[2:3, :] + p[6:7, :]
    a1 = p[1:2, :] + p[5:6, :]
    a3 = p[3:4, :] + p[7:8, :]
    return (a0 + a2) + (a1 + a3)


def _mm(a, b):
    return lax.dot_general(a, b, (((0,), (0,)), ((), ())),
                           preferred_element_type=jnp.float32)


def _vq_kernel(x_ref, e_ref, out_ref, idx_ref):
    e = e_ref[...]
    en2 = jnp.sum(e * e, axis=0, keepdims=True)
    e_hi, e_mid, e_lo = _split3(e)
    B = x_ref.shape[0]
    N = x_ref.shape[2]
    K = e.shape[1]
    for b in range(B):
        xb = x_ref[b]
        x_hi, x_mid, _ = _split3(xb)
        prod = (_mm(x_hi, e_hi) + _mm(x_hi, e_mid)) + _mm(x_mid, e_hi)
        scores = en2 - 2.0 * prod
        kiota = lax.broadcasted_iota(jnp.int32, (N, K), 1)
        i1 = jnp.argmin(scores, axis=-1).astype(jnp.int32)
        masked = jnp.where(kiota == i1[:, None], jnp.inf, scores)
        i2 = jnp.argmin(masked, axis=-1).astype(jnp.int32)

        niota = lax.broadcasted_iota(jnp.int32, (K, N), 0)
        oh1 = (niota == i1[None, :]).astype(jnp.bfloat16)
        oh2 = (niota == i2[None, :]).astype(jnp.bfloat16)
        def _gather(oh):
            return ((jnp.dot(e_hi, oh, preferred_element_type=jnp.float32)
                     + jnp.dot(e_mid, oh, preferred_element_type=jnp.float32))
                    + jnp.dot(e_lo, oh, preferred_element_type=jnp.float32))

        g1 = _gather(oh1)
        g2 = _gather(oh2)

        d1 = jnp.sqrt(_exact_d2(xb, g1))
        d2 = jnp.sqrt(_exact_d2(xb, g2))
        i1r = i1[None, :]
        i2r = i2[None, :]
        swap = (d2 < d1) | ((d2 == d1) & (i2r < i1r))
        idx_ref[b] = jnp.where(swap, i2r, i1r)[0]
        out_ref[b] = jnp.where(swap, g2, g1)


def kernel(feature_vectors, embeddings):
    B, D, N = feature_vectors.shape
    out, idx = pl.pallas_call(
        _vq_kernel,
        out_shape=(
            jax.ShapeDtypeStruct((B, D, N), jnp.float32),
            jax.ShapeDtypeStruct((B, N), jnp.int32),
        ),
    )(feature_vectors, embeddings)
    return out, idx


# speedup vs baseline: 5.9230x; 1.2088x over previous
import jax
import jax.numpy as jnp
from jax import lax
from jax.experimental import pallas as pl


def _split3(a):
    hi = a.astype(jnp.bfloat16)
    r = a - hi.astype(jnp.float32)
    mid = r.astype(jnp.bfloat16)
    lo = (r - mid.astype(jnp.float32)).astype(jnp.bfloat16)
    return hi, mid, lo


def _exact_d2(xb, g):
    diff = xb - g
    t = diff * diff
    p = t[0:8, :]
    for gi in range(1, 8):
        p = p + t[8 * gi:8 * gi + 8, :]
    a0 = p[0:1, :] + p[4:5, :]
    a2 = p[2:3, :] + p[6:7, :]
    a1 = p[1:2, :] + p[5:6, :]
    a3 = p[3:4, :] + p[7:8, :]
    return (a0 + a2) + (a1 + a3)


def _mm(a, b):
    return lax.dot_general(a, b, (((0,), (0,)), ((), ())),
                           preferred_element_type=jnp.float32)


def _vq_kernel(x_ref, e_ref, out_ref, idx_ref):
    e = e_ref[...]
    en2 = jnp.sum(e * e, axis=0, keepdims=True)
    e_hi, e_mid, e_lo = _split3(e)
    B = x_ref.shape[0]
    N = x_ref.shape[2]
    K = e.shape[1]
    kiota = lax.broadcasted_iota(jnp.int32, (N, K), 1)
    niota = lax.broadcasted_iota(jnp.int32, (K, N), 0)

    def _gather(oh):
        return ((jnp.dot(e_hi, oh, preferred_element_type=jnp.float32)
                 + jnp.dot(e_mid, oh, preferred_element_type=jnp.float32))
                + jnp.dot(e_lo, oh, preferred_element_type=jnp.float32))

    scores_l, i1_l, i2_l, g_l = [], [], [], []
    for b in range(B):
        xb = x_ref[b]
        x_hi, x_mid, _ = _split3(xb)
        prod = (_mm(x_hi, e_hi) + _mm(x_hi, e_mid)) + _mm(x_mid, e_hi)
        scores_l.append(en2 - 2.0 * prod)
    for b in range(B):
        scores = scores_l[b]
        i1 = jnp.argmin(scores, axis=-1).astype(jnp.int32)
        masked = jnp.where(kiota == i1[:, None], jnp.inf, scores)
        i1_l.append(i1)
        i2_l.append(jnp.argmin(masked, axis=-1).astype(jnp.int32))
    for b in range(B):
        oh1 = (niota == i1_l[b][None, :]).astype(jnp.bfloat16)
        oh2 = (niota == i2_l[b][None, :]).astype(jnp.bfloat16)
        g_l.append((_gather(oh1), _gather(oh2)))
    for b in range(B):
        xb = x_ref[b]
        g1, g2 = g_l[b]
        d1 = jnp.sqrt(_exact_d2(xb, g1))
        d2 = jnp.sqrt(_exact_d2(xb, g2))
        i1r = i1_l[b][None, :]
        i2r = i2_l[b][None, :]
        swap = (d2 < d1) | ((d2 == d1) & (i2r < i1r))
        idx_ref[b] = jnp.where(swap, i2r, i1r)[0]
        out_ref[b] = jnp.where(swap, g2, g1)


def kernel(feature_vectors, embeddings):
    B, D, N = feature_vectors.shape
    out, idx = pl.pallas_call(
        _vq_kernel,
        out_shape=(
            jax.ShapeDtypeStruct((B, D, N), jnp.float32),
            jax.ShapeDtypeStruct((B, N), jnp.int32),
        ),
    )(feature_vectors, embeddings)
    return out, idx


# speedup vs baseline: 7.8444x; 1.3244x over previous
import jax
import jax.numpy as jnp
from jax import lax
from jax.experimental import pallas as pl


def _split3(a):
    hi = a.astype(jnp.bfloat16)
    r = a - hi.astype(jnp.float32)
    mid = r.astype(jnp.bfloat16)
    lo = (r - mid.astype(jnp.float32)).astype(jnp.bfloat16)
    return hi, mid, lo


def _exact_d2(xb, g):
    diff = xb - g
    t = diff * diff
    p = t[0:8, :]
    for gi in range(1, 8):
        p = p + t[8 * gi:8 * gi + 8, :]
    a0 = p[0:1, :] + p[4:5, :]
    a2 = p[2:3, :] + p[6:7, :]
    a1 = p[1:2, :] + p[5:6, :]
    a3 = p[3:4, :] + p[7:8, :]
    return (a0 + a2) + (a1 + a3)


def _mm(a, b):
    return lax.dot_general(a, b, (((0,), (0,)), ((), ())),
                           preferred_element_type=jnp.float32)


def _vq_kernel(x_ref, e_ref, out_ref, idx_ref):
    e = e_ref[...]
    en2 = jnp.sum(e * e, axis=0, keepdims=True)
    e_hi, e_mid, e_lo = _split3(e)
    B = x_ref.shape[0]
    N = x_ref.shape[2]
    K = e.shape[1]
    kiota = lax.broadcasted_iota(jnp.int32, (N, K), 1)
    niota2 = lax.broadcasted_iota(jnp.int32, (K, 512), 0)
    e3s = jnp.concatenate([e_hi, e_mid, e_hi], axis=0)
    e3g = jnp.concatenate([e_hi, e_mid, e_lo], axis=0)
    zpad = jnp.zeros((1, 256 - N), jnp.int32)

    scores_l, i1_l, i2_l, g_l = [], [], [], []
    for b in range(B):
        xb = x_ref[b]
        x_hi, x_mid, _ = _split3(xb)
        x3 = jnp.concatenate([x_hi, x_hi, x_mid], axis=0)
        scores_l.append(en2 - 2.0 * _mm(x3, e3s))
    for b in range(B):
        scores = scores_l[b]
        i1 = jnp.argmin(scores, axis=-1).astype(jnp.int32)
        masked = jnp.where(kiota == i1[:, None], jnp.inf, scores)
        i1_l.append(i1)
        i2_l.append(jnp.argmin(masked, axis=-1).astype(jnp.int32))
    for b in range(B):
        icat = jnp.concatenate(
            [i1_l[b][None, :], zpad, i2_l[b][None, :], zpad], axis=1)
        oh = (niota2 == icat).astype(jnp.bfloat16)
        gcat = jnp.dot(e3g, oh, preferred_element_type=jnp.float32)
        g1 = ((gcat[0:64, 0:N] + gcat[64:128, 0:N])
              + gcat[128:192, 0:N])
        g2 = ((gcat[0:64, 256:256 + N] + gcat[64:128, 256:256 + N])
              + gcat[128:192, 256:256 + N])
        g_l.append((g1, g2))
    for b in range(B):
        xb = x_ref[b]
        g1, g2 = g_l[b]
        d1 = jnp.sqrt(_exact_d2(xb, g1))
        d2 = jnp.sqrt(_exact_d2(xb, g2))
        i1r = i1_l[b][None, :]
        i2r = i2_l[b][None, :]
        swap = (d2 < d1) | ((d2 == d1) & (i2r < i1r))
        idx_ref[b] = jnp.where(swap, i2r, i1r)[0]
        out_ref[b] = jnp.where(swap, g2, g1)


def kernel(feature_vectors, embeddings):
    B, D, N = feature_vectors.shape
    out, idx = pl.pallas_call(
        _vq_kernel,
        out_shape=(
            jax.ShapeDtypeStruct((B, D, N), jnp.float32),
            jax.ShapeDtypeStruct((B, N), jnp.int32),
        ),
    )(feature_vectors, embeddings)
    return out, idx
